# baseline (speedup 1.0000x reference)
import functools

import jax
import jax.numpy as jnp
import numpy as np
from jax import lax
from jax.experimental import pallas as pl
from jax.experimental.pallas import tpu as pltpu
from jax.experimental.pallas import tpu_sc as plsc


def _fold_tables(tableT, W1, b1, Wr, br):
    D, V = tableT.shape
    H1 = W1.shape[1]
    H2 = Wr.shape[1]
    BLK = 2048
    nblk = pl.cdiv(V, BLK)
    Vp = nblk * BLK

    wpad = jnp.zeros((D, 32), jnp.float32)
    wpad = wpad.at[:, 0:H1].set(W1).at[:, H1:H1 + H2].set(Wr)
    bpad = jnp.zeros((32,), jnp.float32)
    bpad = bpad.at[0:H1].set(b1).at[H1:H1 + H2].set(br)

    def body(t_ref, w_ref, b_ref, o_ref):
        x = lax.dot_general(
            t_ref[...], w_ref[...], (((0,), (0,)), ((), ())),
            preferred_element_type=jnp.float32)
        g = jax.nn.sigmoid(x + b_ref[...][None, :])
        row = pl.program_id(0) * BLK + lax.broadcasted_iota(jnp.int32, g.shape, 0)
        lane = lax.broadcasted_iota(jnp.int32, g.shape, 1)
        g = jnp.where((row == 0) & (lane < H1), 0.0, g)
        o_ref[:, 0:32] = g

    packed = pl.pallas_call(
        body,
        grid=(nblk,),
        in_specs=[
            pl.BlockSpec((D, BLK), lambda i: (0, i)),
            pl.BlockSpec((D, 32), lambda i: (0, 0)),
            pl.BlockSpec((32,), lambda i: (0,)),
        ],
        out_specs=pl.BlockSpec((BLK, 128), lambda i: (i, 0)),
        out_shape=jax.ShapeDtypeStruct((Vp, 128), jnp.float32),
    )(tableT, wpad, bpad)
    return packed.reshape(8 * Vp, 16)


def _sc_pool_and_target(seq8, tgt81, gview):
    B, L = seq8.shape
    info = plsc.get_sparse_core_info()
    NC, NS = info.num_cores, info.num_subcores
    NW = NC * NS
    PB = B // NW
    C = 4
    NCH = PB // C
    CL = C * L
    NSUB = pl.cdiv(CL, 128)
    CLP = NSUB * 128

    nfull = L // 16
    rem = L % 16
    n1 = min(L, 128)

    tok = np.arange(CLP)
    inner = np.where(tok < CL, tok // L, PB)[None, None, :]
    sidx_np = (np.arange(NCH)[:, None, None] * C + inner)
    sidx_np = np.minimum(sidx_np, PB).reshape(1, NCH, NSUB, 128)
    sidx_np = sidx_np + (np.arange(NS) * (PB + 1)).reshape(NS, 1, 1, 1)
    sidx_const = jnp.asarray(sidx_np.astype(np.int32))

    mesh = plsc.VectorSubcoreMesh(core_axis_name="c", subcore_axis_name="s")

    @functools.partial(
        pl.kernel,
        out_type=[
            jax.ShapeDtypeStruct((B, 16), jnp.float32),
            jax.ShapeDtypeStruct((B, 16), jnp.float32),
            jax.ShapeDtypeStruct((B, 16), jnp.float32),
        ],
        mesh=mesh,
        compiler_params=pltpu.CompilerParams(use_tc_tiling_on_sc=False),
        scratch_types=[
            pltpu.VMEM((PB, L), jnp.int32),
            pltpu.VMEM((2, CLP, 16), jnp.float32),
            pltpu.VMEM((NCH, NSUB, 128), jnp.int32),
            pltpu.VMEM_SHARED((NS * (PB + 1), 16), jnp.float32),
            pltpu.VMEM((PB, 16), jnp.float32),
            pltpu.VMEM((PB,), jnp.int32),
            pltpu.VMEM((PB, 16), jnp.float32),
            pltpu.SemaphoreType.DMA,
            pltpu.SemaphoreType.DMA,
            pltpu.SemaphoreType.DMA,
            pltpu.SemaphoreType.DMA,
        ],
    )
    def k(seq_h, tgt_h, sidx_h, g_h, psum_h, pcnt_h, th_h,
          seq_v, buf_v, sidx_v, pacc_sh, pcnt_v, tgt_v, th_v,
          sem_ga, sem_gb, sem_s, sem_t):
        sid = lax.axis_index("s")
        wid = sid * NC + lax.axis_index("c")
        base = wid * PB
        abase = sid * (PB + 1)
        pltpu.sync_copy(seq_h.at[pl.ds(base, PB)], seq_v)
        pltpu.sync_copy(tgt_h.at[pl.ds(base, PB)], tgt_v)
        pltpu.sync_copy(sidx_h.at[sid], sidx_v)
        tcopy = pltpu.async_copy(g_h.at[tgt_v], th_v, sem_t)

        def zero_body(b, carry):
            buf_v[0, b, :] = jnp.zeros((16,), jnp.float32)
            return carry

        lax.fori_loop(0, PB + 1, zero_body, 0)
        pltpu.sync_copy(buf_v.at[0, pl.ds(0, PB + 1)],
                        pacc_sh.at[pl.ds(abase, PB + 1)])

        gsems = (sem_ga, sem_gb)

        def issue(ck, slot):
            sem = gsems[slot]
            for r in range(C):
                b = ck * C + r
                pltpu.async_copy(
                    g_h.at[seq_v.at[b, pl.ds(0, n1)]],
                    buf_v.at[slot, pl.ds(r * L, n1)], sem)
                if L > 128:
                    pltpu.async_copy(
                        g_h.at[seq_v.at[b, pl.ds(128, L - 128)]],
                        buf_v.at[slot, pl.ds(r * L + 128, L - 128)], sem)

        def drain(slot):
            sem = gsems[slot]
            for r in range(C):
                pltpu.make_async_copy(
                    g_h.at[pl.ds(0, n1)],
                    buf_v.at[slot, pl.ds(r * L, n1)], sem).wait()
                if L > 128:
                    pltpu.make_async_copy(
                        g_h.at[pl.ds(0, L - 128)],
                        buf_v.at[slot, pl.ds(r * L + 128, L - 128)],
                        sem).wait()

        def process(ck, slot):
            drain(slot)
            scat = [
                pltpu.async_copy(
                    buf_v.at[slot, pl.ds(j * 128, 128)],
                    pacc_sh.at[sidx_v.at[ck, j]], sem_s, add=True)
                for j in range(NSUB)
            ]
            for r in range(C):
                b = ck * C + r
                cntv = jnp.zeros((16,), jnp.int32)
                for kk in range(nfull):
                    v = seq_v[b, pl.ds(kk * 16, 16)]
                    cntv = cntv + jnp.where(v != 0, 1, 0)
                if rem:
                    v = seq_v[b, pl.ds(L - 16, 16)]
                    lane = lax.iota(jnp.int32, 16)
                    cntv = cntv + jnp.where(
                        (v != 0) & (lane >= (16 - rem)), 1, 0)
                pcnt_v[b, :] = cntv.astype(jnp.float32)
            for c in scat:
                c.wait()

        issue(0, 0)
        issue(1, 1)

        def body(i, carry):
            k0 = 2 * i
            process(k0, 0)
            issue(jnp.minimum(k0 + 2, NCH - 1), 0)
            process(k0 + 1, 1)
            issue(jnp.minimum(k0 + 3, NCH - 1), 1)
            return carry

        lax.fori_loop(0, NCH // 2, body, 0)
        drain(0)
        drain(1)
        tcopy.wait()
        pltpu.sync_copy(pacc_sh.at[pl.ds(abase, PB)],
                        psum_h.at[pl.ds(base, PB)])
        pltpu.sync_copy(pcnt_v, pcnt_h.at[pl.ds(base, PB)])
        pltpu.sync_copy(th_v, th_h.at[pl.ds(base, PB)])

    return k(seq8, tgt81, sidx_const, gview)


def _tail(psum, pcnt, th, W2, b2):
    B = psum.shape[0]
    H1 = W2.shape[0]
    H2 = W2.shape[1]

    def body(ps_ref, pc_ref, th_ref, w2_ref, b2_ref, o_ref):
        acc = ps_ref[...]
        cnt = jnp.sum(pc_ref[...], axis=1, keepdims=True)
        pooled = acc / jnp.maximum(cnt, 1.0)
        sh = jax.nn.sigmoid(
            jnp.dot(pooled, w2_ref[...], preferred_element_type=jnp.float32)
            + b2_ref[...][None, :])
        o_ref[...] = jnp.sum(sh * th_ref[...][:, 0:H2], axis=1, keepdims=True)

    return pl.pallas_call(
        body,
        in_specs=[
            pl.BlockSpec((B, 16), lambda: (0, 0)),
            pl.BlockSpec((B, 16), lambda: (0, 0)),
            pl.BlockSpec((B, 16), lambda: (0, 0)),
            pl.BlockSpec((H1, H2), lambda: (0, 0)),
            pl.BlockSpec((H2,), lambda: (0,)),
        ],
        out_specs=pl.BlockSpec((B, 1), lambda: (0, 0)),
        out_shape=jax.ShapeDtypeStruct((B, 1), jnp.float32),
    )(psum, pcnt, th, W2, b2)


def kernel(seq, target, table, W1, b1, W2, b2, Wr, br):
    B, L = seq.shape
    gview = _fold_tables(table.T, W1, b1, Wr, br)
    seq8 = seq * 8
    tgt81 = target.reshape(B) * 8 + 1
    psum, pcnt, th = _sc_pool_and_target(seq8, tgt81, gview)
    out = _tail(psum, pcnt, th, W2, b2)
    return out.reshape(B, target.shape[1] * target.shape[2])


# speedup vs baseline: 1.0262x; 1.0262x over previous
import functools

import jax
import jax.numpy as jnp
import numpy as np
from jax import lax
from jax.experimental import pallas as pl
from jax.experimental.pallas import tpu as pltpu
from jax.experimental.pallas import tpu_sc as plsc


def _fold_tables(tableT, W1, b1, Wr, br):
    D, V = tableT.shape
    H1 = W1.shape[1]
    H2 = Wr.shape[1]
    BLK = 2048
    nblk = pl.cdiv(V, BLK)
    Vp = nblk * BLK

    wpad = jnp.zeros((D, 32), jnp.float32)
    wpad = wpad.at[:, 0:H1].set(W1).at[:, H1:H1 + H2].set(Wr)
    bpad = jnp.zeros((32,), jnp.float32)
    bpad = bpad.at[0:H1].set(b1).at[H1:H1 + H2].set(br)

    def body(t_ref, w_ref, b_ref, o_ref):
        x = lax.dot_general(
            t_ref[...], w_ref[...], (((0,), (0,)), ((), ())),
            preferred_element_type=jnp.float32)
        g = jax.nn.sigmoid(x + b_ref[...][None, :])
        row = pl.program_id(0) * BLK + lax.broadcasted_iota(jnp.int32, g.shape, 0)
        lane = lax.broadcasted_iota(jnp.int32, g.shape, 1)
        g = jnp.where((row == 0) & (lane < H1), 0.0, g)
        o_ref[:, 0:32] = g

    packed = pl.pallas_call(
        body,
        grid=(nblk,),
        in_specs=[
            pl.BlockSpec((D, BLK), lambda i: (0, i)),
            pl.BlockSpec((D, 32), lambda i: (0, 0)),
            pl.BlockSpec((32,), lambda i: (0,)),
        ],
        out_specs=pl.BlockSpec((BLK, 128), lambda i: (i, 0)),
        out_shape=jax.ShapeDtypeStruct((Vp, 128), jnp.float32),
    )(tableT, wpad, bpad)
    return packed.reshape(8 * Vp, 16)


def _sc_pool_and_target(seq, tgt, gview):
    B, L = seq.shape
    info = plsc.get_sparse_core_info()
    NC, NS = info.num_cores, info.num_subcores
    NW = NC * NS
    PB = B // NW
    C = 4
    NCH = PB // C
    CL = C * L
    NSUB = pl.cdiv(CL, 128)
    CLP = NSUB * 128

    nfull = L // 16
    rem = L % 16
    n1 = min(L, 128)

    tok = np.arange(CLP)
    inner = np.where(tok < CL, tok // L, PB)[None, None, :]
    sidx_np = (np.arange(NCH)[:, None, None] * C + inner)
    sidx_np = np.minimum(sidx_np, PB).reshape(1, NCH, NSUB, 128)
    sidx_np = sidx_np + (np.arange(NS) * (PB + 1)).reshape(NS, 1, 1, 1)
    sidx_const = jnp.asarray(sidx_np.astype(np.int32))

    mesh = plsc.VectorSubcoreMesh(core_axis_name="c", subcore_axis_name="s")

    @functools.partial(
        pl.kernel,
        out_type=[
            jax.ShapeDtypeStruct((B, 16), jnp.float32),
            jax.ShapeDtypeStruct((B, 16), jnp.float32),
            jax.ShapeDtypeStruct((B, 16), jnp.float32),
        ],
        mesh=mesh,
        compiler_params=pltpu.CompilerParams(use_tc_tiling_on_sc=False),
        scratch_types=[
            pltpu.VMEM((PB, L), jnp.int32),
            pltpu.VMEM((2, CLP, 16), jnp.float32),
            pltpu.VMEM((NCH, NSUB, 128), jnp.int32),
            pltpu.VMEM_SHARED((NS * (PB + 1), 16), jnp.float32),
            pltpu.VMEM((PB, 16), jnp.float32),
            pltpu.VMEM((PB,), jnp.int32),
            pltpu.VMEM((PB, 16), jnp.float32),
            pltpu.SemaphoreType.DMA,
            pltpu.SemaphoreType.DMA,
            pltpu.SemaphoreType.DMA,
            pltpu.SemaphoreType.DMA,
        ],
    )
    def k(seq_h, tgt_h, sidx_h, g_h, psum_h, pcnt_h, th_h,
          seq_v, buf_v, sidx_v, pacc_sh, pcnt_v, tgt_v, th_v,
          sem_ga, sem_gb, sem_s, sem_t):
        sid = lax.axis_index("s")
        wid = sid * NC + lax.axis_index("c")
        base = wid * PB
        abase = sid * (PB + 1)
        pltpu.sync_copy(seq_h.at[pl.ds(base, PB)], seq_v)
        pltpu.sync_copy(tgt_h.at[pl.ds(base, PB)], tgt_v)
        pltpu.sync_copy(sidx_h.at[sid], sidx_v)

        for m in range(PB // 16):
            tgt_v[pl.ds(16 * m, 16)] = tgt_v[pl.ds(16 * m, 16)] * 8 + 1
        tcopy = pltpu.async_copy(g_h.at[tgt_v], th_v, sem_t)

        lane16 = lax.iota(jnp.int32, 16)

        def scale_body(b, carry):
            for kk in range(L // 16):
                seq_v[b, pl.ds(kk * 16, 16)] = seq_v[b, pl.ds(kk * 16, 16)] * 8
            if rem:
                v = seq_v[b, pl.ds(L - 16, 16)]
                seq_v[b, pl.ds(L - 16, 16)] = jnp.where(
                    lane16 < (16 - rem), v, v * 8)
            return carry

        lax.fori_loop(0, PB, scale_body, 0)

        def zero_body(b, carry):
            buf_v[0, b, :] = jnp.zeros((16,), jnp.float32)
            return carry

        lax.fori_loop(0, PB + 1, zero_body, 0)
        pltpu.sync_copy(buf_v.at[0, pl.ds(0, PB + 1)],
                        pacc_sh.at[pl.ds(abase, PB + 1)])

        gsems = (sem_ga, sem_gb)

        def issue(ck, slot):
            sem = gsems[slot]
            for r in range(C):
                b = ck * C + r
                pltpu.async_copy(
                    g_h.at[seq_v.at[b, pl.ds(0, n1)]],
                    buf_v.at[slot, pl.ds(r * L, n1)], sem)
                if L > 128:
                    pltpu.async_copy(
                        g_h.at[seq_v.at[b, pl.ds(128, L - 128)]],
                        buf_v.at[slot, pl.ds(r * L + 128, L - 128)], sem)

        def drain(slot):
            sem = gsems[slot]
            for r in range(C):
                pltpu.make_async_copy(
                    g_h.at[pl.ds(0, n1)],
                    buf_v.at[slot, pl.ds(r * L, n1)], sem).wait()
                if L > 128:
                    pltpu.make_async_copy(
                        g_h.at[pl.ds(0, L - 128)],
                        buf_v.at[slot, pl.ds(r * L + 128, L - 128)],
                        sem).wait()

        def process(ck, slot):
            drain(slot)
            scat = [
                pltpu.async_copy(
                    buf_v.at[slot, pl.ds(j * 128, 128)],
                    pacc_sh.at[sidx_v.at[ck, j]], sem_s, add=True)
                for j in range(NSUB)
            ]
            for r in range(C):
                b = ck * C + r
                cntv = jnp.zeros((16,), jnp.int32)
                for kk in range(nfull):
                    v = seq_v[b, pl.ds(kk * 16, 16)]
                    cntv = cntv + jnp.where(v != 0, 1, 0)
                if rem:
                    v = seq_v[b, pl.ds(L - 16, 16)]
                    lane = lax.iota(jnp.int32, 16)
                    cntv = cntv + jnp.where(
                        (v != 0) & (lane >= (16 - rem)), 1, 0)
                pcnt_v[b, :] = cntv.astype(jnp.float32)
            for c in scat:
                c.wait()

        issue(0, 0)
        issue(1, 1)

        def body(i, carry):
            k0 = 2 * i
            process(k0, 0)
            issue(jnp.minimum(k0 + 2, NCH - 1), 0)
            process(k0 + 1, 1)
            issue(jnp.minimum(k0 + 3, NCH - 1), 1)
            return carry

        lax.fori_loop(0, NCH // 2, body, 0)
        drain(0)
        drain(1)
        tcopy.wait()
        pltpu.sync_copy(pacc_sh.at[pl.ds(abase, PB)],
                        psum_h.at[pl.ds(base, PB)])
        pltpu.sync_copy(pcnt_v, pcnt_h.at[pl.ds(base, PB)])
        pltpu.sync_copy(th_v, th_h.at[pl.ds(base, PB)])

    return k(seq, tgt, sidx_const, gview)


def _tail(psum, pcnt, th, W2, b2):
    B = psum.shape[0]
    H1 = W2.shape[0]
    H2 = W2.shape[1]

    def body(ps_ref, pc_ref, th_ref, w2_ref, b2_ref, o_ref):
        acc = ps_ref[...]
        cnt = jnp.sum(pc_ref[...], axis=1, keepdims=True)
        pooled = acc / jnp.maximum(cnt, 1.0)
        sh = jax.nn.sigmoid(
            jnp.dot(pooled, w2_ref[...], preferred_element_type=jnp.float32)
            + b2_ref[...][None, :])
        o_ref[...] = jnp.sum(sh * th_ref[...][:, 0:H2], axis=1, keepdims=True)

    return pl.pallas_call(
        body,
        in_specs=[
            pl.BlockSpec((B, 16), lambda: (0, 0)),
            pl.BlockSpec((B, 16), lambda: (0, 0)),
            pl.BlockSpec((B, 16), lambda: (0, 0)),
            pl.BlockSpec((H1, H2), lambda: (0, 0)),
            pl.BlockSpec((H2,), lambda: (0,)),
        ],
        out_specs=pl.BlockSpec((B, 1), lambda: (0, 0)),
        out_shape=jax.ShapeDtypeStruct((B, 1), jnp.float32),
    )(psum, pcnt, th, W2, b2)


def kernel(seq, target, table, W1, b1, W2, b2, Wr, br):
    B, L = seq.shape
    gview = _fold_tables(table.T, W1, b1, Wr, br)
    psum, pcnt, th = _sc_pool_and_target(seq, target.reshape(B), gview)
    out = _tail(psum, pcnt, th, W2, b2)
    return out.reshape(B, target.shape[1] * target.shape[2])


# speedup vs baseline: 1.0546x; 1.0276x over previous
import functools

import jax
import jax.numpy as jnp
import numpy as np
from jax import lax
from jax.experimental import pallas as pl
from jax.experimental.pallas import tpu as pltpu
from jax.experimental.pallas import tpu_sc as plsc


def _fold_tables(tableT, W1, b1, Wr, br):
    D, V = tableT.shape
    H1 = W1.shape[1]
    H2 = Wr.shape[1]
    BLK = 2048
    nblk = pl.cdiv(V, BLK)
    Vp = nblk * BLK

    wpad = jnp.zeros((D, 32), jnp.float32)
    wpad = wpad.at[:, 0:H1].set(W1).at[:, H1:H1 + H2].set(Wr)
    bpad = jnp.zeros((32,), jnp.float32)
    bpad = bpad.at[0:H1].set(b1).at[H1:H1 + H2].set(br)

    def body(t_ref, w_ref, b_ref, o_ref):
        x = lax.dot_general(
            t_ref[...], w_ref[...], (((0,), (0,)), ((), ())),
            preferred_element_type=jnp.float32)
        g = jax.nn.sigmoid(x + b_ref[...][None, :])
        row = pl.program_id(0) * BLK + lax.broadcasted_iota(jnp.int32, g.shape, 0)
        lane = lax.broadcasted_iota(jnp.int32, g.shape, 1)
        g = jnp.where((row == 0) & (lane < H1), 0.0, g)
        o_ref[:, 0:32] = g

    packed = pl.pallas_call(
        body,
        grid=(nblk,),
        in_specs=[
            pl.BlockSpec((D, BLK), lambda i: (0, i)),
            pl.BlockSpec((D, 32), lambda i: (0, 0)),
            pl.BlockSpec((32,), lambda i: (0,)),
        ],
        out_specs=pl.BlockSpec((BLK, 128), lambda i: (i, 0)),
        out_shape=jax.ShapeDtypeStruct((Vp, 128), jnp.float32),
    )(tableT, wpad, bpad)
    return packed.reshape(8 * Vp, 16)


def _sc_pool_and_target(seq, tgt, gview):
    B, L = seq.shape
    info = plsc.get_sparse_core_info()
    NC, NS = info.num_cores, info.num_subcores
    NW = NC * NS
    PB = B // NW
    C = 4
    NCH = PB // C
    CL = C * L
    NSUB = pl.cdiv(CL, 128)
    CLP = NSUB * 128

    nfull = L // 16
    rem = L % 16
    n1 = min(L, 128)

    tok = np.arange(CLP)
    inner = np.where(tok < CL, tok // L, PB)[None, None, :]
    sidx_np = (np.arange(NCH)[:, None, None] * C + inner)
    sidx_np = np.minimum(sidx_np, PB).reshape(1, NCH, NSUB, 128)
    sidx_np = sidx_np + (np.arange(NS) * (PB + 1)).reshape(NS, 1, 1, 1)
    sidx_const = jnp.asarray(sidx_np.astype(np.int32))

    mesh = plsc.VectorSubcoreMesh(core_axis_name="c", subcore_axis_name="s")

    @functools.partial(
        pl.kernel,
        out_type=[
            jax.ShapeDtypeStruct((B, 16), jnp.float32),
            jax.ShapeDtypeStruct((B, 16), jnp.float32),
            jax.ShapeDtypeStruct((B, 16), jnp.float32),
            jax.ShapeDtypeStruct((B, 16), jnp.float32),
        ],
        mesh=mesh,
        compiler_params=pltpu.CompilerParams(use_tc_tiling_on_sc=False),
        scratch_types=[
            pltpu.VMEM((PB, L), jnp.int32),
            pltpu.VMEM((2, CLP, 16), jnp.float32),
            pltpu.VMEM((NCH, NSUB, 128), jnp.int32),
            pltpu.VMEM_SHARED((NS * (PB + 1), 16), jnp.float32),
            pltpu.VMEM((PB, 16), jnp.float32),
            pltpu.VMEM((PB, 16), jnp.float32),
            pltpu.VMEM((PB,), jnp.int32),
            pltpu.VMEM((PB, 16), jnp.float32),
            pltpu.SemaphoreType.DMA,
            pltpu.SemaphoreType.DMA,
            pltpu.SemaphoreType.DMA,
            pltpu.SemaphoreType.DMA,
        ],
    )
    def k(seq_h, tgt_h, sidx_h, g_h, psum_h, psum2_h, pcnt_h, th_h,
          seq_v, buf_v, sidx_v, pacc_sh, psum2_v, pcnt_v, tgt_v, th_v,
          sem_ga, sem_gb, sem_s, sem_t):
        sid = lax.axis_index("s")
        wid = sid * NC + lax.axis_index("c")
        base = wid * PB
        abase = sid * (PB + 1)
        pltpu.sync_copy(seq_h.at[pl.ds(base, PB)], seq_v)
        pltpu.sync_copy(tgt_h.at[pl.ds(base, PB)], tgt_v)
        pltpu.sync_copy(sidx_h.at[sid], sidx_v)

        for m in range(PB // 16):
            tgt_v[pl.ds(16 * m, 16)] = tgt_v[pl.ds(16 * m, 16)] * 8 + 1
        tcopy = pltpu.async_copy(g_h.at[tgt_v], th_v, sem_t)

        lane16 = lax.iota(jnp.int32, 16)

        def scale_body(b, carry):
            for kk in range(L // 16):
                seq_v[b, pl.ds(kk * 16, 16)] = seq_v[b, pl.ds(kk * 16, 16)] * 8
            if rem:
                v = seq_v[b, pl.ds(L - 16, 16)]
                seq_v[b, pl.ds(L - 16, 16)] = jnp.where(
                    lane16 < (16 - rem), v, v * 8)
            return carry

        lax.fori_loop(0, PB, scale_body, 0)

        def zero_body(b, carry):
            buf_v[0, b, :] = jnp.zeros((16,), jnp.float32)
            return carry

        lax.fori_loop(0, PB + 1, zero_body, 0)
        pltpu.sync_copy(buf_v.at[0, pl.ds(0, PB + 1)],
                        pacc_sh.at[pl.ds(abase, PB + 1)])

        def zero2_body(b, carry):
            psum2_v[b, :] = jnp.zeros((16,), jnp.float32)
            return carry

        lax.fori_loop(0, PB, zero2_body, 0)

        gsems = (sem_ga, sem_gb)

        def issue(ck, slot):
            sem = gsems[slot]
            for r in range(C):
                b = ck * C + r
                pltpu.async_copy(
                    g_h.at[seq_v.at[b, pl.ds(0, n1)]],
                    buf_v.at[slot, pl.ds(r * L, n1)], sem)
                if L > 128:
                    pltpu.async_copy(
                        g_h.at[seq_v.at[b, pl.ds(128, L - 128)]],
                        buf_v.at[slot, pl.ds(r * L + 128, L - 128)], sem)

        def drain(slot):
            sem = gsems[slot]
            for r in range(C):
                pltpu.make_async_copy(
                    g_h.at[pl.ds(0, n1)],
                    buf_v.at[slot, pl.ds(r * L, n1)], sem).wait()
                if L > 128:
                    pltpu.make_async_copy(
                        g_h.at[pl.ds(0, L - 128)],
                        buf_v.at[slot, pl.ds(r * L + 128, L - 128)],
                        sem).wait()

        def process(ck, slot):
            drain(slot)
            scat = [
                pltpu.async_copy(
                    buf_v.at[slot, pl.ds(j * 128, 128)],
                    pacc_sh.at[sidx_v.at[ck, j]], sem_s, add=True)
                for j in range(NSUB)
            ]
            for r in range(C):
                b = ck * C + r
                cntv = jnp.zeros((16,), jnp.int32)
                for kk in range(nfull):
                    v = seq_v[b, pl.ds(kk * 16, 16)]
                    cntv = cntv + jnp.where(v != 0, 1, 0)
                if rem:
                    v = seq_v[b, pl.ds(L - 16, 16)]
                    lane = lax.iota(jnp.int32, 16)
                    cntv = cntv + jnp.where(
                        (v != 0) & (lane >= (16 - rem)), 1, 0)
                pcnt_v[b, :] = cntv.astype(jnp.float32)
            for c in scat:
                c.wait()

        def process_vec(ck, slot):
            drain(slot)
            for r in range(C):
                b = ck * C + r
                cntv = jnp.zeros((16,), jnp.int32)
                for kk in range(nfull):
                    v = seq_v[b, pl.ds(kk * 16, 16)]
                    cntv = cntv + jnp.where(v != 0, 1, 0)
                if rem:
                    v = seq_v[b, pl.ds(L - 16, 16)]
                    lane = lax.iota(jnp.int32, 16)
                    cntv = cntv + jnp.where(
                        (v != 0) & (lane >= (16 - rem)), 1, 0)
                pcnt_v[b, :] = cntv.astype(jnp.float32)

                zero = jnp.zeros((16,), jnp.float32)
                accs = [zero, zero, zero, zero]
                for j in range(L):
                    accs[j % 4] = accs[j % 4] + buf_v[slot, r * L + j, :]
                psum2_v[b, :] = (accs[0] + accs[1]) + (accs[2] + accs[3])

        issue(0, 0)
        issue(1, 1)

        def body(i, carry):
            k0 = 2 * i
            process(k0, 0)
            issue(jnp.minimum(k0 + 2, NCH - 1), 0)
            process_vec(k0 + 1, 1)
            issue(jnp.minimum(k0 + 3, NCH - 1), 1)
            return carry

        lax.fori_loop(0, NCH // 2, body, 0)
        drain(0)
        drain(1)
        tcopy.wait()
        pltpu.sync_copy(pacc_sh.at[pl.ds(abase, PB)],
                        psum_h.at[pl.ds(base, PB)])
        pltpu.sync_copy(psum2_v, psum2_h.at[pl.ds(base, PB)])
        pltpu.sync_copy(pcnt_v, pcnt_h.at[pl.ds(base, PB)])
        pltpu.sync_copy(th_v, th_h.at[pl.ds(base, PB)])

    return k(seq, tgt, sidx_const, gview)


def _tail(psum, psum2, pcnt, th, W2, b2):
    B = psum.shape[0]
    H1 = W2.shape[0]
    H2 = W2.shape[1]

    def body(ps_ref, ps2_ref, pc_ref, th_ref, w2_ref, b2_ref, o_ref):
        acc = ps_ref[...] + ps2_ref[...]
        cnt = jnp.sum(pc_ref[...], axis=1, keepdims=True)
        pooled = acc / jnp.maximum(cnt, 1.0)
        sh = jax.nn.sigmoid(
            jnp.dot(pooled, w2_ref[...], preferred_element_type=jnp.float32)
            + b2_ref[...][None, :])
        o_ref[...] = jnp.sum(sh * th_ref[...][:, 0:H2], axis=1, keepdims=True)

    return pl.pallas_call(
        body,
        in_specs=[
            pl.BlockSpec((B, 16), lambda: (0, 0)),
            pl.BlockSpec((B, 16), lambda: (0, 0)),
            pl.BlockSpec((B, 16), lambda: (0, 0)),
            pl.BlockSpec((B, 16), lambda: (0, 0)),
            pl.BlockSpec((H1, H2), lambda: (0, 0)),
            pl.BlockSpec((H2,), lambda: (0,)),
        ],
        out_specs=pl.BlockSpec((B, 1), lambda: (0, 0)),
        out_shape=jax.ShapeDtypeStruct((B, 1), jnp.float32),
    )(psum, psum2, pcnt, th, W2, b2)


def kernel(seq, target, table, W1, b1, W2, b2, Wr, br):
    B, L = seq.shape
    gview = _fold_tables(table.T, W1, b1, Wr, br)
    psum, psum2, pcnt, th = _sc_pool_and_target(seq, target.reshape(B), gview)
    out = _tail(psum, psum2, pcnt, th, W2, b2)
    return out.reshape(B, target.shape[1] * target.shape[2])
